# baseline (speedup 1.0000x reference)
import functools

import jax
import jax.numpy as jnp
from jax import lax
from jax.experimental import pallas as pl
from jax.experimental.pallas import tpu as pltpu
from jax.experimental.pallas import tpu_sc as plsc

NV = 40962
NV_PREV = 10242
NF = 81920
BS = 2
C = 128
D = BS * C

_NC, _NSC = 2, 16
NW = _NC * _NSC
NVP = 41472
NVPL = 43008
VPW = NVPL // NW

FCH = 8
FNCH = (NF // NW) // FCH
LCH = 16
LNCH = VPW // LCH
VCH = 16
VNCH = VPW // VCH

_MESH = dict(core_axis_name="c", subcore_axis_name="s",
             num_cores=_NC, num_subcores=_NSC)


def _wid():
    return lax.axis_index("s") * _NC + lax.axis_index("c")


def _emit_pass(table, idx_hbm, w_hbm, out_hbm, bufs, sems, rch, nch,
               compute_rows):
    idxb = bufs[0:2]
    wb = bufs[2:4]
    gb = bufs[4:6]
    ob = bufs[6:8]
    smi = sems[0:2]
    smw = sems[2:4]
    smg = sems[4:6]
    smo = sems[6:8]
    c0 = _wid() * nch

    pltpu.sync_copy(idx_hbm.at[c0], idxb[0])
    pltpu.async_copy(idx_hbm.at[c0 + 1], idxb[1], smi[1])
    pltpu.async_copy(w_hbm.at[c0], wb[0], smw[0])
    pltpu.async_copy(table.at[idxb[0]], gb[0], smg[0])

    def pair(p, carry):
        for b in range(2):
            ci = 2 * p + b
            c = c0 + ci
            s, s1 = b, 1 - b
            pltpu.make_async_copy(table.at[idxb[s]], gb[s], smg[s]).wait()

            @pl.when(ci + 2 < nch)
            def _():
                pltpu.async_copy(idx_hbm.at[c + 2], idxb[s], smi[s])

            @pl.when(ci + 1 < nch)
            def _():
                pltpu.make_async_copy(idx_hbm.at[c + 1], idxb[s1], smi[s1]).wait()
                pltpu.async_copy(w_hbm.at[c + 1], wb[s1], smw[s1])
                pltpu.async_copy(table.at[idxb[s1]], gb[s1], smg[s1])

            pltpu.make_async_copy(w_hbm.at[c], wb[s], smw[s]).wait()

            @pl.when(ci >= 2)
            def _():
                pltpu.make_async_copy(
                    ob[s], out_hbm.at[pl.ds(c * rch, rch)], smo[s]).wait()

            compute_rows(gb[s], wb[s], ob[s])
            pltpu.async_copy(ob[s], out_hbm.at[pl.ds(c * rch, rch)], smo[s])
        return carry

    lax.fori_loop(0, nch // 2, pair, 0)
    pltpu.make_async_copy(ob[0], out_hbm.at[pl.ds(c0 * rch, rch)], smo[0]).wait()
    pltpu.make_async_copy(ob[1], out_hbm.at[pl.ds(c0 * rch, rch)], smo[1]).wait()


def _face_rows(gb, wvb, ob):
    def face(fb, carry):
        g0 = fb * 9
        wv = wvb[pl.ds(fb * 32, 16)]
        ev = wvb[pl.ds(fb * 32 + 16, 16)]
        w = [wv[j] for j in range(9)]
        e = [ev[j] for j in range(6)]
        for t in range(D // 16):
            sl = pl.ds(t * 16, 16)
            s0 = w[0] * gb[g0 + 0, sl] + w[1] * gb[g0 + 1, sl] + w[2] * gb[g0 + 2, sl]
            s1 = w[3] * gb[g0 + 3, sl] + w[4] * gb[g0 + 4, sl] + w[5] * gb[g0 + 5, sl]
            s2 = w[6] * gb[g0 + 6, sl] + w[7] * gb[g0 + 7, sl] + w[8] * gb[g0 + 8, sl]
            ob[fb, sl] = e[0] * s0 + e[1] * s1 + e[2] * s2
            ob[fb, pl.ds(D + t * 16, 16)] = e[3] * s0 + e[4] * s1 + e[5] * s2
        return carry

    lax.fori_loop(0, FCH, face, 0)


def _lap_rows(gb, wvb, ob):
    def vert(vb, carry):
        g0 = vb * 7
        wv = wvb[pl.ds(vb * 16, 16)]
        w = [wv[j] for j in range(7)]
        for t in range(D // 16):
            sl = pl.ds(t * 16, 16)
            acc = w[0] * gb[g0 + 0, sl]
            for j in range(1, 7):
                acc = acc + w[j] * gb[g0 + j, sl]
            ob[vb, sl] = acc
        return carry

    lax.fori_loop(0, LCH, vert, 0)


def _f2v_rows(gb, wvb, ob):
    def vert(vb, carry):
        g0 = vb * 6
        wv = wvb[pl.ds(vb * 16, 16)]
        w = [wv[j] for j in range(6)]
        for t in range(2 * D // 16):
            sl = pl.ds(t * 16, 16)
            acc = w[0] * gb[g0 + 0, sl]
            for j in range(1, 6):
                acc = acc + w[j] * gb[g0 + j, sl]
            ob[vb, sl] = acc
        return carry

    lax.fori_loop(0, VCH, vert, 0)


def _buf_set(nidx, wlen, outd, rch):
    return [
        pltpu.VMEM((nidx,), jnp.int32),
        pltpu.VMEM((nidx,), jnp.int32),
        pltpu.VMEM((wlen,), jnp.float32),
        pltpu.VMEM((wlen,), jnp.float32),
        pltpu.VMEM((nidx, D), jnp.float32),
        pltpu.VMEM((nidx, D), jnp.float32),
        pltpu.VMEM((rch, outd), jnp.float32),
        pltpu.VMEM((rch, outd), jnp.float32),
    ]


def _face_lap_body(xt, gidx, gw, lidx, lw, faces2, lap, *scr):
    fbufs = scr[0:8]
    lbufs = scr[8:16]
    sems = scr[16:24]
    _emit_pass(xt, gidx, gw, faces2, fbufs, sems, FCH, FNCH, _face_rows)
    _emit_pass(xt, lidx, lw, lap, lbufs, sems, LCH, LNCH, _lap_rows)


def _sc_face_lap(xt, gidx, gw, lidx, lw):
    scr = (_buf_set(FCH * 9, FCH * 32, 2 * D, FCH)
           + _buf_set(LCH * 7, LCH * 16, D, LCH)
           + [pltpu.SemaphoreType.DMA] * 8)
    return pl.kernel(
        _face_lap_body,
        out_type=(jax.ShapeDtypeStruct((NF, 2 * D), jnp.float32),
                  jax.ShapeDtypeStruct((NVPL, D), jnp.float32)),
        mesh=plsc.VectorSubcoreMesh(**_MESH),
        scratch_types=scr,
    )(xt, gidx, gw, lidx, lw)


def _f2v_body(faces2, fidx, fw, gv, *scr):
    _emit_pass(faces2, fidx, fw, gv, scr[0:8], scr[8:16], VCH, VNCH, _f2v_rows)


def _sc_f2v(faces2, fidx, fw):
    scr = _buf_set(VCH * 6, VCH * 16, 2 * D, VCH)[:4] + [
        pltpu.VMEM((VCH * 6, 2 * D), jnp.float32),
        pltpu.VMEM((VCH * 6, 2 * D), jnp.float32),
        pltpu.VMEM((VCH, 2 * D), jnp.float32),
        pltpu.VMEM((VCH, 2 * D), jnp.float32),
    ] + [pltpu.SemaphoreType.DMA] * 8
    return pl.kernel(
        _f2v_body,
        out_type=jax.ShapeDtypeStruct((NVPL, 2 * D), jnp.float32),
        mesh=plsc.VectorSubcoreMesh(**_MESH),
        scratch_types=scr,
    )(faces2, fidx, fw)


_BLK = 512


def _combine_body(xt_ref, lap_ref, gv_ref, cs_ref, out_ref):
    cs = cs_ref[...]
    for b in range(BS):
        x = xt_ref[:, b * C:(b + 1) * C]
        l = lap_ref[:, b * C:(b + 1) * C]
        e = gv_ref[:, b * C:(b + 1) * C]
        n = gv_ref[:, D + b * C:D + (b + 1) * C]
        acc = (jnp.dot(x, cs[0:C], preferred_element_type=jnp.float32)
               + jnp.dot(l, cs[C:2 * C], preferred_element_type=jnp.float32)
               + jnp.dot(e, cs[2 * C:3 * C], preferred_element_type=jnp.float32)
               + jnp.dot(n, cs[3 * C:4 * C], preferred_element_type=jnp.float32))
        out_ref[b] = acc.T


def _tc_combine(xtp, lap, gv, cstack):
    return pl.pallas_call(
        _combine_body,
        grid=(NVP // _BLK,),
        in_specs=[
            pl.BlockSpec((_BLK, D), lambda i: (i, 0)),
            pl.BlockSpec((_BLK, D), lambda i: (i, 0)),
            pl.BlockSpec((_BLK, 2 * D), lambda i: (i, 0)),
            pl.BlockSpec((4 * C, C), lambda i: (0, 0)),
        ],
        out_specs=pl.BlockSpec((BS, C, _BLK), lambda i: (0, 0, i)),
        out_shape=jax.ShapeDtypeStruct((BS, C, NVP), jnp.float32),
    )(xtp, lap, gv, cstack)


def kernel(input, coeffs, G_rows, G_cols, G_vals, L_rows, L_cols, L_vals,
           F_rows, F_cols, F_vals, NS, EW):
    f32 = jnp.float32
    x = jnp.concatenate(
        [input, jnp.ones((BS, C, NV - NV_PREV), dtype=input.dtype)], axis=-1)
    xt = x.transpose(2, 0, 1).reshape(NV, D)
    xtp = jnp.concatenate([xt, jnp.zeros((NVP - NV, D), dtype=f32)], axis=0)

    gcols9 = G_cols.reshape(3, NF, 3).transpose(1, 0, 2).reshape(NF, 9)
    gvals9 = G_vals.reshape(3, NF, 3).transpose(1, 0, 2).reshape(NF, 9)
    gidx = gcols9.reshape(NF // FCH, FCH * 9)
    gw = jnp.concatenate(
        [gvals9, jnp.zeros((NF, 7), f32), EW, NS, jnp.zeros((NF, 10), f32)],
        axis=1).reshape(NF // FCH, FCH * 32)

    zpad = NVPL - NV
    lidx = jnp.concatenate(
        [L_cols.reshape(NV, 7), jnp.zeros((zpad, 7), jnp.int32)], axis=0
    ).reshape(NVPL // LCH, LCH * 7)
    lw = jnp.concatenate(
        [L_vals.reshape(NV, 7), jnp.zeros((NV, 9), f32)], axis=1)
    lw = jnp.concatenate([lw, jnp.zeros((zpad, 16), f32)], axis=0
                         ).reshape(NVPL // LCH, LCH * 16)

    fidx = jnp.concatenate(
        [F_cols.reshape(NV, 6), jnp.zeros((zpad, 6), jnp.int32)], axis=0
    ).reshape(NVPL // VCH, VCH * 6)
    fw = jnp.concatenate(
        [F_vals.reshape(NV, 6), jnp.zeros((NV, 10), f32)], axis=1)
    fw = jnp.concatenate([fw, jnp.zeros((zpad, 16), f32)], axis=0
                         ).reshape(NVPL // VCH, VCH * 16)

    cstack = coeffs.reshape(C, 4, C).transpose(1, 0, 2).reshape(4 * C, C)

    faces2, lap = _sc_face_lap(xtp, gidx, gw, lidx, lw)
    gv = _sc_f2v(faces2, fidx, fw)
    out = _tc_combine(xtp, lap, gv, cstack)
    return out[:, :, :NV]


# speedup vs baseline: 1.2400x; 1.2400x over previous
import functools

import jax
import jax.numpy as jnp
from jax import lax
from jax.experimental import pallas as pl
from jax.experimental.pallas import tpu as pltpu
from jax.experimental.pallas import tpu_sc as plsc

NV = 40962
NV_PREV = 10242
NF = 81920
BS = 2
C = 128
D = BS * C

_NC, _NSC = 2, 16
NW = _NC * _NSC
NVP = 41472
NVPL = NVP
VPW = NVPL // NW

FCH = 8
FNCH = (NF // NW) // FCH
LCH = 8
LNCH = VPW // LCH
VCH = 8
VNCH = VPW // VCH

_MESH = dict(core_axis_name="c", subcore_axis_name="s",
             num_cores=_NC, num_subcores=_NSC)


def _wid():
    return lax.axis_index("s") * _NC + lax.axis_index("c")


def _emit_pass(table, idx_hbm, w_hbm, out_hbm, bufs, sems, rch, nch,
               compute_rows):
    idxb = bufs[0:2]
    wb = bufs[2:4]
    gb = bufs[4:6]
    ob = bufs[6:8]
    smi = sems[0:2]
    smw = sems[2:4]
    smg = sems[4:6]
    smo = sems[6:8]
    c0 = _wid() * nch

    pltpu.sync_copy(idx_hbm.at[c0], idxb[0])
    pltpu.async_copy(idx_hbm.at[c0 + 1], idxb[1], smi[1])
    pltpu.async_copy(w_hbm.at[c0], wb[0], smw[0])
    pltpu.async_copy(table.at[idxb[0]], gb[0], smg[0])

    def pair(p, carry):
        for b in range(2):
            ci = 2 * p + b
            c = c0 + ci
            s, s1 = b, 1 - b
            pltpu.make_async_copy(table.at[idxb[s]], gb[s], smg[s]).wait()

            @pl.when(ci + 2 < nch)
            def _():
                pltpu.async_copy(idx_hbm.at[c + 2], idxb[s], smi[s])

            @pl.when(ci + 1 < nch)
            def _():
                pltpu.make_async_copy(idx_hbm.at[c + 1], idxb[s1], smi[s1]).wait()
                pltpu.async_copy(w_hbm.at[c + 1], wb[s1], smw[s1])
                pltpu.async_copy(table.at[idxb[s1]], gb[s1], smg[s1])

            pltpu.make_async_copy(w_hbm.at[c], wb[s], smw[s]).wait()

            @pl.when(ci >= 2)
            def _():
                pltpu.make_async_copy(
                    ob[s], out_hbm.at[pl.ds(c * rch, rch)], smo[s]).wait()

            compute_rows(gb[s], wb[s], ob[s])
            pltpu.async_copy(ob[s], out_hbm.at[pl.ds(c * rch, rch)], smo[s])
        return carry

    lax.fori_loop(0, nch // 2, pair, 0)
    pltpu.make_async_copy(ob[0], out_hbm.at[pl.ds(c0 * rch, rch)], smo[0]).wait()
    pltpu.make_async_copy(ob[1], out_hbm.at[pl.ds(c0 * rch, rch)], smo[1]).wait()


def _face_rows(gb, wvb, ob):
    def face(fb, carry):
        g0 = fb * 9
        wv = wvb[pl.ds(fb * 32, 16)]
        ev = wvb[pl.ds(fb * 32 + 16, 16)]
        w = [wv[j] for j in range(9)]
        e = [ev[j] for j in range(6)]
        for t in range(D // 16):
            sl = pl.ds(t * 16, 16)
            s0 = w[0] * gb[g0 + 0, sl] + w[1] * gb[g0 + 1, sl] + w[2] * gb[g0 + 2, sl]
            s1 = w[3] * gb[g0 + 3, sl] + w[4] * gb[g0 + 4, sl] + w[5] * gb[g0 + 5, sl]
            s2 = w[6] * gb[g0 + 6, sl] + w[7] * gb[g0 + 7, sl] + w[8] * gb[g0 + 8, sl]
            ob[fb, sl] = e[0] * s0 + e[1] * s1 + e[2] * s2
            ob[fb, pl.ds(D + t * 16, 16)] = e[3] * s0 + e[4] * s1 + e[5] * s2
        return carry

    lax.fori_loop(0, FCH, face, 0)


def _lap_rows(gb, wvb, ob):
    def vert(vb, carry):
        g0 = vb * 7
        wv = wvb[pl.ds(vb * 16, 16)]
        w = [wv[j] for j in range(7)]
        for t in range(D // 16):
            sl = pl.ds(t * 16, 16)
            acc = w[0] * gb[g0 + 0, sl]
            for j in range(1, 7):
                acc = acc + w[j] * gb[g0 + j, sl]
            ob[vb, sl] = acc
        return carry

    lax.fori_loop(0, LCH, vert, 0)


def _f2v_rows(gb, wvb, ob):
    def vert(vb, carry):
        g0 = vb * 6
        wv = wvb[pl.ds(vb * 16, 16)]
        w = [wv[j] for j in range(6)]
        for t in range(2 * D // 16):
            sl = pl.ds(t * 16, 16)
            acc = w[0] * gb[g0 + 0, sl]
            for j in range(1, 6):
                acc = acc + w[j] * gb[g0 + j, sl]
            ob[vb, sl] = acc
        return carry

    lax.fori_loop(0, VCH, vert, 0)


def _buf_set(nidx, wlen, outd, rch):
    return [
        pltpu.VMEM((nidx,), jnp.int32),
        pltpu.VMEM((nidx,), jnp.int32),
        pltpu.VMEM((wlen,), jnp.float32),
        pltpu.VMEM((wlen,), jnp.float32),
        pltpu.VMEM((nidx, D), jnp.float32),
        pltpu.VMEM((nidx, D), jnp.float32),
        pltpu.VMEM((rch, outd), jnp.float32),
        pltpu.VMEM((rch, outd), jnp.float32),
    ]


def _face_lap_body(xt, gidx, gw, lidx, lw, faces2, lap, *scr):
    fbufs = scr[0:8]
    lbufs = scr[8:16]
    sems = scr[16:24]
    _emit_pass(xt, gidx, gw, faces2, fbufs, sems, FCH, FNCH, _face_rows)
    _emit_pass(xt, lidx, lw, lap, lbufs, sems, LCH, LNCH, _lap_rows)


def _sc_face_lap(xt, gidx, gw, lidx, lw):
    scr = (_buf_set(FCH * 9, FCH * 32, 2 * D, FCH)
           + _buf_set(LCH * 7, LCH * 16, D, LCH)
           + [pltpu.SemaphoreType.DMA] * 8)
    return pl.kernel(
        _face_lap_body,
        out_type=(jax.ShapeDtypeStruct((NF, 2 * D), jnp.float32),
                  jax.ShapeDtypeStruct((NVPL, D), jnp.float32)),
        mesh=plsc.VectorSubcoreMesh(**_MESH),
        scratch_types=scr,
    )(xt, gidx, gw, lidx, lw)


def _f2v_body(faces2, fidx, fw, gv, *scr):
    _emit_pass(faces2, fidx, fw, gv, scr[0:8], scr[8:16], VCH, VNCH, _f2v_rows)


def _sc_f2v(faces2, fidx, fw):
    scr = _buf_set(VCH * 6, VCH * 16, 2 * D, VCH)[:4] + [
        pltpu.VMEM((VCH * 6, 2 * D), jnp.float32),
        pltpu.VMEM((VCH * 6, 2 * D), jnp.float32),
        pltpu.VMEM((VCH, 2 * D), jnp.float32),
        pltpu.VMEM((VCH, 2 * D), jnp.float32),
    ] + [pltpu.SemaphoreType.DMA] * 8
    return pl.kernel(
        _f2v_body,
        out_type=jax.ShapeDtypeStruct((NVPL, 2 * D), jnp.float32),
        mesh=plsc.VectorSubcoreMesh(**_MESH),
        scratch_types=scr,
    )(faces2, fidx, fw)


_BLK = 512


def _combine_body(xt_ref, lap_ref, gv_ref, cs_ref, out_ref):
    cs = cs_ref[...]
    for b in range(BS):
        x = xt_ref[:, b * C:(b + 1) * C]
        l = lap_ref[:, b * C:(b + 1) * C]
        e = gv_ref[:, b * C:(b + 1) * C]
        n = gv_ref[:, D + b * C:D + (b + 1) * C]
        acc = (jnp.dot(x, cs[0:C], preferred_element_type=jnp.float32)
               + jnp.dot(l, cs[C:2 * C], preferred_element_type=jnp.float32)
               + jnp.dot(e, cs[2 * C:3 * C], preferred_element_type=jnp.float32)
               + jnp.dot(n, cs[3 * C:4 * C], preferred_element_type=jnp.float32))
        out_ref[b] = acc.T


def _tc_combine(xtp, lap, gv, cstack):
    return pl.pallas_call(
        _combine_body,
        grid=(NVP // _BLK,),
        in_specs=[
            pl.BlockSpec((_BLK, D), lambda i: (i, 0)),
            pl.BlockSpec((_BLK, D), lambda i: (i, 0)),
            pl.BlockSpec((_BLK, 2 * D), lambda i: (i, 0)),
            pl.BlockSpec((4 * C, C), lambda i: (0, 0)),
        ],
        out_specs=pl.BlockSpec((BS, C, _BLK), lambda i: (0, 0, i)),
        out_shape=jax.ShapeDtypeStruct((BS, C, NVP), jnp.float32),
    )(xtp, lap, gv, cstack)


def kernel(input, coeffs, G_rows, G_cols, G_vals, L_rows, L_cols, L_vals,
           F_rows, F_cols, F_vals, NS, EW):
    f32 = jnp.float32
    x = jnp.concatenate(
        [input, jnp.ones((BS, C, NV - NV_PREV), dtype=input.dtype)], axis=-1)
    xt = x.transpose(2, 0, 1).reshape(NV, D)
    xtp = jnp.concatenate([xt, jnp.zeros((NVP - NV, D), dtype=f32)], axis=0)

    gcols9 = G_cols.reshape(3, NF, 3).transpose(1, 0, 2).reshape(NF, 9)
    gvals9 = G_vals.reshape(3, NF, 3).transpose(1, 0, 2).reshape(NF, 9)
    gidx = gcols9.reshape(NF // FCH, FCH * 9)
    gw = jnp.concatenate(
        [gvals9, jnp.zeros((NF, 7), f32), EW, NS, jnp.zeros((NF, 10), f32)],
        axis=1).reshape(NF // FCH, FCH * 32)

    zpad = NVPL - NV
    lidx = jnp.concatenate(
        [L_cols.reshape(NV, 7), jnp.zeros((zpad, 7), jnp.int32)], axis=0
    ).reshape(NVPL // LCH, LCH * 7)
    lw = jnp.concatenate(
        [L_vals.reshape(NV, 7), jnp.zeros((NV, 9), f32)], axis=1)
    lw = jnp.concatenate([lw, jnp.zeros((zpad, 16), f32)], axis=0
                         ).reshape(NVPL // LCH, LCH * 16)

    fidx = jnp.concatenate(
        [F_cols.reshape(NV, 6), jnp.zeros((zpad, 6), jnp.int32)], axis=0
    ).reshape(NVPL // VCH, VCH * 6)
    fw = jnp.concatenate(
        [F_vals.reshape(NV, 6), jnp.zeros((NV, 10), f32)], axis=1)
    fw = jnp.concatenate([fw, jnp.zeros((zpad, 16), f32)], axis=0
                         ).reshape(NVPL // VCH, VCH * 16)

    cstack = coeffs.reshape(C, 4, C).transpose(1, 0, 2).reshape(4 * C, C)

    faces2, lap = _sc_face_lap(xtp, gidx, gw, lidx, lw)
    gv = _sc_f2v(faces2, fidx, fw)
    out = _tc_combine(xtp, lap, gv, cstack)
    return out[:, :, :NV]


# speedup vs baseline: 1.3674x; 1.1027x over previous
import functools

import jax
import jax.numpy as jnp
from jax import lax
from jax.experimental import pallas as pl
from jax.experimental.pallas import tpu as pltpu
from jax.experimental.pallas import tpu_sc as plsc

NV = 40962
NV_PREV = 10242
NF = 81920
BS = 2
C = 128
D = BS * C

_NC, _NSC = 2, 16
NW = _NC * _NSC
NVP = 41472
VPW = NVP // NW

FCH = 8
FNCH = (NF // NW) // FCH
LCH = 8
LNCH = VPW // LCH
VCH = 8
VNCH = VPW // VCH

_MESH = dict(core_axis_name="c", subcore_axis_name="s",
             num_cores=_NC, num_subcores=_NSC)


def _wid():
    return lax.axis_index("s") * _NC + lax.axis_index("c")


def _emit_pass(table, idx_hbm, w_hbm, out_hbm, bufs, sems, rch, nch,
               compute_rows):
    idxb = bufs[0:2]
    wb = bufs[2:4]
    gb = bufs[4:6]
    ob = bufs[6:8]
    smi = sems[0:2]
    smw = sems[2:4]
    smg = sems[4:6]
    smo = sems[6:8]
    c0 = _wid() * nch

    pltpu.sync_copy(idx_hbm.at[c0], idxb[0])
    pltpu.async_copy(idx_hbm.at[c0 + 1], idxb[1], smi[1])
    pltpu.async_copy(w_hbm.at[c0], wb[0], smw[0])
    pltpu.async_copy(table.at[idxb[0]], gb[0], smg[0])

    def pair(p, carry):
        for b in range(2):
            ci = 2 * p + b
            c = c0 + ci
            s, s1 = b, 1 - b
            pltpu.make_async_copy(table.at[idxb[s]], gb[s], smg[s]).wait()

            @pl.when(ci + 2 < nch)
            def _():
                pltpu.async_copy(idx_hbm.at[c + 2], idxb[s], smi[s])

            @pl.when(ci + 1 < nch)
            def _():
                pltpu.make_async_copy(idx_hbm.at[c + 1], idxb[s1], smi[s1]).wait()
                pltpu.async_copy(w_hbm.at[c + 1], wb[s1], smw[s1])
                pltpu.async_copy(table.at[idxb[s1]], gb[s1], smg[s1])

            pltpu.make_async_copy(w_hbm.at[c], wb[s], smw[s]).wait()

            @pl.when(ci >= 2)
            def _():
                pltpu.make_async_copy(
                    ob[s], out_hbm.at[pl.ds(c * rch, rch)], smo[s]).wait()

            compute_rows(gb[s], wb[s], ob[s])
            pltpu.async_copy(ob[s], out_hbm.at[pl.ds(c * rch, rch)], smo[s])
        return carry

    lax.fori_loop(0, nch // 2, pair, 0)
    pltpu.make_async_copy(ob[0], out_hbm.at[pl.ds(c0 * rch, rch)], smo[0]).wait()
    pltpu.make_async_copy(ob[1], out_hbm.at[pl.ds(c0 * rch, rch)], smo[1]).wait()


def _face_rows(gb, wvb, ob):
    @plsc.parallel_loop(0, FCH)
    def face(fb):
        g0 = fb * 9
        wv = wvb[pl.ds(fb * 32, 16)]
        ev = wvb[pl.ds(fb * 32 + 16, 16)]
        w = [wv[j] for j in range(9)]
        e = [ev[j] for j in range(6)]
        for t in range(D // 16):
            sl = pl.ds(t * 16, 16)
            s0 = w[0] * gb[g0 + 0, sl] + w[1] * gb[g0 + 1, sl] + w[2] * gb[g0 + 2, sl]
            s1 = w[3] * gb[g0 + 3, sl] + w[4] * gb[g0 + 4, sl] + w[5] * gb[g0 + 5, sl]
            s2 = w[6] * gb[g0 + 6, sl] + w[7] * gb[g0 + 7, sl] + w[8] * gb[g0 + 8, sl]
            ob[fb, sl] = e[0] * s0 + e[1] * s1 + e[2] * s2
            ob[fb, pl.ds(D + t * 16, 16)] = e[3] * s0 + e[4] * s1 + e[5] * s2


def _lap_rows(gb, wvb, ob):
    @plsc.parallel_loop(0, LCH)
    def vert(vb):
        g0 = vb * 7
        wv = wvb[pl.ds(vb * 16, 16)]
        w = [wv[j] for j in range(7)]
        for t in range(D // 16):
            sl = pl.ds(t * 16, 16)
            acc = w[0] * gb[g0 + 0, sl]
            for j in range(1, 7):
                acc = acc + w[j] * gb[g0 + j, sl]
            ob[vb, sl] = acc


def _f2v_rows(gb, wvb, ob):
    @plsc.parallel_loop(0, VCH)
    def vert(vb):
        g0 = vb * 6
        wv = wvb[pl.ds(vb * 16, 16)]
        w = [wv[j] for j in range(6)]
        for t in range(2 * D // 16):
            sl = pl.ds(t * 16, 16)
            acc = w[0] * gb[g0 + 0, sl]
            for j in range(1, 6):
                acc = acc + w[j] * gb[g0 + j, sl]
            ob[vb, sl] = acc


def _buf_set(nidx, wlen, outd, rch):
    return [
        pltpu.VMEM((nidx,), jnp.int32),
        pltpu.VMEM((nidx,), jnp.int32),
        pltpu.VMEM((wlen,), jnp.float32),
        pltpu.VMEM((wlen,), jnp.float32),
        pltpu.VMEM((nidx, D), jnp.float32),
        pltpu.VMEM((nidx, D), jnp.float32),
        pltpu.VMEM((rch, outd), jnp.float32),
        pltpu.VMEM((rch, outd), jnp.float32),
    ]


def _face_body(xt, gidx, gw, faces2, *scr):
    _emit_pass(xt, gidx, gw, faces2, scr[0:8], scr[8:16], FCH, FNCH, _face_rows)


def _sc_face(xt, gidx, gw):
    scr = _buf_set(FCH * 9, FCH * 32, 2 * D, FCH) + [pltpu.SemaphoreType.DMA] * 8
    return pl.kernel(
        _face_body,
        out_type=jax.ShapeDtypeStruct((NF, 2 * D), jnp.float32),
        mesh=plsc.VectorSubcoreMesh(**_MESH),
        scratch_types=scr,
    )(xt, gidx, gw)


def _lap_body(xt, lidx, lw, lap, *scr):
    _emit_pass(xt, lidx, lw, lap, scr[0:8], scr[8:16], LCH, LNCH, _lap_rows)


def _sc_lap(xt, lidx, lw):
    scr = _buf_set(LCH * 7, LCH * 16, D, LCH) + [pltpu.SemaphoreType.DMA] * 8
    return pl.kernel(
        _lap_body,
        out_type=jax.ShapeDtypeStruct((NVP, D), jnp.float32),
        mesh=plsc.VectorSubcoreMesh(**_MESH),
        scratch_types=scr,
    )(xt, lidx, lw)


def _f2v_body(faces2, fidx, fw, gv, *scr):
    _emit_pass(faces2, fidx, fw, gv, scr[0:8], scr[8:16], VCH, VNCH, _f2v_rows)


def _sc_f2v(faces2, fidx, fw):
    scr = [
        pltpu.VMEM((VCH * 6,), jnp.int32),
        pltpu.VMEM((VCH * 6,), jnp.int32),
        pltpu.VMEM((VCH * 16,), jnp.float32),
        pltpu.VMEM((VCH * 16,), jnp.float32),
        pltpu.VMEM((VCH * 6, 2 * D), jnp.float32),
        pltpu.VMEM((VCH * 6, 2 * D), jnp.float32),
        pltpu.VMEM((VCH, 2 * D), jnp.float32),
        pltpu.VMEM((VCH, 2 * D), jnp.float32),
    ] + [pltpu.SemaphoreType.DMA] * 8
    return pl.kernel(
        _f2v_body,
        out_type=jax.ShapeDtypeStruct((NVP, 2 * D), jnp.float32),
        mesh=plsc.VectorSubcoreMesh(**_MESH),
        scratch_types=scr,
    )(faces2, fidx, fw)


_BLK = 512


def _combine1_body(xt_ref, lap_ref, cs_ref, out_ref):
    cs = cs_ref[...]
    for b in range(BS):
        x = xt_ref[:, b * C:(b + 1) * C]
        l = lap_ref[:, b * C:(b + 1) * C]
        acc = (jnp.dot(x, cs[0:C], preferred_element_type=jnp.float32)
               + jnp.dot(l, cs[C:2 * C], preferred_element_type=jnp.float32))
        out_ref[b] = acc.T


def _combine2_body(part_ref, gv_ref, cs_ref, out_ref):
    cs = cs_ref[...]
    for b in range(BS):
        e = gv_ref[:, b * C:(b + 1) * C]
        n = gv_ref[:, D + b * C:D + (b + 1) * C]
        acc = (jnp.dot(e, cs[0:C], preferred_element_type=jnp.float32)
               + jnp.dot(n, cs[C:2 * C], preferred_element_type=jnp.float32))
        out_ref[b] = part_ref[b] + acc.T


def _tc_combine1(xtp, lap, cs01):
    return pl.pallas_call(
        _combine1_body,
        grid=(NVP // _BLK,),
        in_specs=[
            pl.BlockSpec((_BLK, D), lambda i: (i, 0)),
            pl.BlockSpec((_BLK, D), lambda i: (i, 0)),
            pl.BlockSpec((2 * C, C), lambda i: (0, 0)),
        ],
        out_specs=pl.BlockSpec((BS, C, _BLK), lambda i: (0, 0, i)),
        out_shape=jax.ShapeDtypeStruct((BS, C, NVP), jnp.float32),
    )(xtp, lap, cs01)


def _tc_combine2(part, gv, cs23):
    return pl.pallas_call(
        _combine2_body,
        grid=(NVP // _BLK,),
        in_specs=[
            pl.BlockSpec((BS, C, _BLK), lambda i: (0, 0, i)),
            pl.BlockSpec((_BLK, 2 * D), lambda i: (i, 0)),
            pl.BlockSpec((2 * C, C), lambda i: (0, 0)),
        ],
        out_specs=pl.BlockSpec((BS, C, _BLK), lambda i: (0, 0, i)),
        out_shape=jax.ShapeDtypeStruct((BS, C, NVP), jnp.float32),
    )(part, gv, cs23)


def kernel(input, coeffs, G_rows, G_cols, G_vals, L_rows, L_cols, L_vals,
           F_rows, F_cols, F_vals, NS, EW):
    f32 = jnp.float32
    x = jnp.concatenate(
        [input, jnp.ones((BS, C, NV - NV_PREV), dtype=input.dtype)], axis=-1)
    xt = x.transpose(2, 0, 1).reshape(NV, D)
    xtp = jnp.concatenate([xt, jnp.zeros((NVP - NV, D), dtype=f32)], axis=0)

    gcols9 = G_cols.reshape(3, NF, 3).transpose(1, 0, 2).reshape(NF, 9)
    gvals9 = G_vals.reshape(3, NF, 3).transpose(1, 0, 2).reshape(NF, 9)
    gidx = gcols9.reshape(NF // FCH, FCH * 9)
    gw = jnp.concatenate(
        [gvals9, jnp.zeros((NF, 7), f32), EW, NS, jnp.zeros((NF, 10), f32)],
        axis=1).reshape(NF // FCH, FCH * 32)

    zpad = NVP - NV
    lidx = jnp.concatenate(
        [L_cols.reshape(NV, 7), jnp.zeros((zpad, 7), jnp.int32)], axis=0
    ).reshape(NVP // LCH, LCH * 7)
    lw = jnp.concatenate(
        [L_vals.reshape(NV, 7), jnp.zeros((NV, 9), f32)], axis=1)
    lw = jnp.concatenate([lw, jnp.zeros((zpad, 16), f32)], axis=0
                         ).reshape(NVP // LCH, LCH * 16)

    fidx = jnp.concatenate(
        [F_cols.reshape(NV, 6), jnp.zeros((zpad, 6), jnp.int32)], axis=0
    ).reshape(NVP // VCH, VCH * 6)
    fw = jnp.concatenate(
        [F_vals.reshape(NV, 6), jnp.zeros((NV, 10), f32)], axis=1)
    fw = jnp.concatenate([fw, jnp.zeros((zpad, 16), f32)], axis=0
                         ).reshape(NVP // VCH, VCH * 16)

    cstack = coeffs.reshape(C, 4, C).transpose(1, 0, 2).reshape(4 * C, C)
    cs01 = cstack[0:2 * C]
    cs23 = cstack[2 * C:4 * C]

    faces2 = _sc_face(xtp, gidx, gw)
    lap = _sc_lap(xtp, lidx, lw)
    part = _tc_combine1(xtp, lap, cs01)
    gv = _sc_f2v(faces2, fidx, fw)
    out = _tc_combine2(part, gv, cs23)
    return out[:, :, :NV]


# speedup vs baseline: 1.4270x; 1.0436x over previous
import functools

import jax
import jax.numpy as jnp
from jax import lax
from jax.experimental import pallas as pl
from jax.experimental.pallas import tpu as pltpu
from jax.experimental.pallas import tpu_sc as plsc

NV = 40962
NV_PREV = 10242
NF = 81920
BS = 2
C = 128
D = BS * C

_NC, _NSC = 2, 16
NW = _NC * _NSC
NVP = 41472
VPW = NVP // NW

FCH = 8
FNCH = (NF // NW) // FCH
LCH = 8
LNCH = VPW // LCH
VCH = 8
VNCH = VPW // VCH

_MESH = dict(core_axis_name="c", subcore_axis_name="s",
             num_cores=_NC, num_subcores=_NSC)


def _wid():
    return lax.axis_index("s") * _NC + lax.axis_index("c")


def _emit_pass(table, idx_hbm, w_hbm, out_hbm, bufs, sems, rch, nch,
               compute_rows):
    idxb = bufs[0:2]
    wb = bufs[2:4]
    gb = bufs[4:6]
    ob = bufs[6:8]
    smi = sems[0:2]
    smw = sems[2:4]
    smg = sems[4:6]
    smo = sems[6:8]
    c0 = _wid() * nch

    pltpu.sync_copy(idx_hbm.at[c0], idxb[0])
    pltpu.async_copy(idx_hbm.at[c0 + 1], idxb[1], smi[1])
    pltpu.async_copy(w_hbm.at[c0], wb[0], smw[0])
    pltpu.async_copy(table.at[idxb[0]], gb[0], smg[0])

    def pair(p, carry):
        for b in range(2):
            ci = 2 * p + b
            c = c0 + ci
            s, s1 = b, 1 - b
            pltpu.make_async_copy(table.at[idxb[s]], gb[s], smg[s]).wait()

            @pl.when(ci + 2 < nch)
            def _():
                pltpu.async_copy(idx_hbm.at[c + 2], idxb[s], smi[s])

            @pl.when(ci + 1 < nch)
            def _():
                pltpu.make_async_copy(idx_hbm.at[c + 1], idxb[s1], smi[s1]).wait()
                pltpu.async_copy(w_hbm.at[c + 1], wb[s1], smw[s1])
                pltpu.async_copy(table.at[idxb[s1]], gb[s1], smg[s1])

            pltpu.make_async_copy(w_hbm.at[c], wb[s], smw[s]).wait()

            @pl.when(ci >= 2)
            def _():
                pltpu.make_async_copy(
                    ob[s], out_hbm.at[pl.ds(c * rch, rch)], smo[s]).wait()

            compute_rows(gb[s], wb[s], ob[s])
            pltpu.async_copy(ob[s], out_hbm.at[pl.ds(c * rch, rch)], smo[s])
        return carry

    lax.fori_loop(0, nch // 2, pair, 0)
    pltpu.make_async_copy(ob[0], out_hbm.at[pl.ds(c0 * rch, rch)], smo[0]).wait()
    pltpu.make_async_copy(ob[1], out_hbm.at[pl.ds(c0 * rch, rch)], smo[1]).wait()


def _face_rows(gb, wvb, ob):
    @plsc.parallel_loop(0, FCH)
    def face(fb):
        g0 = fb * 9
        wv = wvb[pl.ds(fb * 16, 16)]
        w = [wv[j] for j in range(9)]
        e = [wv[9 + j] for j in range(6)]
        for t in range(D // 16):
            sl = pl.ds(t * 16, 16)
            s0 = w[0] * gb[g0 + 0, sl] + w[1] * gb[g0 + 1, sl] + w[2] * gb[g0 + 2, sl]
            s1 = w[3] * gb[g0 + 3, sl] + w[4] * gb[g0 + 4, sl] + w[5] * gb[g0 + 5, sl]
            s2 = w[6] * gb[g0 + 6, sl] + w[7] * gb[g0 + 7, sl] + w[8] * gb[g0 + 8, sl]
            ob[fb, sl] = e[0] * s0 + e[1] * s1 + e[2] * s2
            ob[fb, pl.ds(D + t * 16, 16)] = e[3] * s0 + e[4] * s1 + e[5] * s2


def _lap_rows(gb, wvb, ob):
    @plsc.parallel_loop(0, LCH)
    def vert(vb):
        g0 = vb * 7
        wv = wvb[pl.ds(vb * 16, 16)]
        w = [wv[j] for j in range(7)]
        for t in range(D // 16):
            sl = pl.ds(t * 16, 16)
            acc = w[0] * gb[g0 + 0, sl]
            for j in range(1, 7):
                acc = acc + w[j] * gb[g0 + j, sl]
            ob[vb, sl] = acc


def _f2v_rows(gb, wvb, ob):
    @plsc.parallel_loop(0, VCH)
    def vert(vb):
        g0 = vb * 6
        wv = wvb[pl.ds(vb * 16, 16)]
        w = [wv[j] for j in range(6)]
        for t in range(2 * D // 16):
            sl = pl.ds(t * 16, 16)
            acc = w[0] * gb[g0 + 0, sl]
            for j in range(1, 6):
                acc = acc + w[j] * gb[g0 + j, sl]
            ob[vb, sl] = acc


def _buf_set(nidx, wlen, outd, rch):
    return [
        pltpu.VMEM((nidx,), jnp.int32),
        pltpu.VMEM((nidx,), jnp.int32),
        pltpu.VMEM((wlen,), jnp.float32),
        pltpu.VMEM((wlen,), jnp.float32),
        pltpu.VMEM((nidx, D), jnp.float32),
        pltpu.VMEM((nidx, D), jnp.float32),
        pltpu.VMEM((rch, outd), jnp.float32),
        pltpu.VMEM((rch, outd), jnp.float32),
    ]


def _face_body(xt, gidx, gw, faces2, *scr):
    _emit_pass(xt, gidx, gw, faces2, scr[0:8], scr[8:16], FCH, FNCH, _face_rows)


def _sc_face(xt, gidx, gw):
    scr = _buf_set(FCH * 9, FCH * 16, 2 * D, FCH) + [pltpu.SemaphoreType.DMA] * 8
    return pl.kernel(
        _face_body,
        out_type=jax.ShapeDtypeStruct((NF, 2 * D), jnp.float32),
        mesh=plsc.VectorSubcoreMesh(**_MESH),
        scratch_types=scr,
    )(xt, gidx, gw)


def _lap_body(xt, lidx, lw, lap, *scr):
    _emit_pass(xt, lidx, lw, lap, scr[0:8], scr[8:16], LCH, LNCH, _lap_rows)


def _sc_lap(xt, lidx, lw):
    scr = _buf_set(LCH * 7, LCH * 16, D, LCH) + [pltpu.SemaphoreType.DMA] * 8
    return pl.kernel(
        _lap_body,
        out_type=jax.ShapeDtypeStruct((NVP, D), jnp.float32),
        mesh=plsc.VectorSubcoreMesh(**_MESH),
        scratch_types=scr,
    )(xt, lidx, lw)


def _f2v_body(faces2, fidx, fw, gv, *scr):
    _emit_pass(faces2, fidx, fw, gv, scr[0:8], scr[8:16], VCH, VNCH, _f2v_rows)


def _sc_f2v(faces2, fidx, fw):
    scr = [
        pltpu.VMEM((VCH * 6,), jnp.int32),
        pltpu.VMEM((VCH * 6,), jnp.int32),
        pltpu.VMEM((VCH * 16,), jnp.float32),
        pltpu.VMEM((VCH * 16,), jnp.float32),
        pltpu.VMEM((VCH * 6, 2 * D), jnp.float32),
        pltpu.VMEM((VCH * 6, 2 * D), jnp.float32),
        pltpu.VMEM((VCH, 2 * D), jnp.float32),
        pltpu.VMEM((VCH, 2 * D), jnp.float32),
    ] + [pltpu.SemaphoreType.DMA] * 8
    return pl.kernel(
        _f2v_body,
        out_type=jax.ShapeDtypeStruct((NVP, 2 * D), jnp.float32),
        mesh=plsc.VectorSubcoreMesh(**_MESH),
        scratch_types=scr,
    )(faces2, fidx, fw)


_BLK = 512


def _xtp_body(xp_ref, out_ref):
    i = pl.program_id(0)
    base = jnp.minimum(i, (NV_PREV // _BLK)) * _BLK
    valid = (i * _BLK + lax.broadcasted_iota(jnp.int32, (_BLK, C), 0)) < NV_PREV
    for b in range(BS):
        vals = xp_ref[b, :, pl.ds(base, _BLK)].T
        out_ref[:, b * C:(b + 1) * C] = jnp.where(valid, vals, 1.0)


def _tc_xtp(xpad):
    return pl.pallas_call(
        _xtp_body,
        grid=(NVP // _BLK,),
        in_specs=[pl.BlockSpec((BS, C, NV_PREV + (_BLK - NV_PREV % _BLK)),
                               lambda i: (0, 0, 0))],
        out_specs=pl.BlockSpec((_BLK, D), lambda i: (i, 0)),
        out_shape=jax.ShapeDtypeStruct((NVP, D), jnp.float32),
    )(xpad)


def _combine1_body(xt_ref, lap_ref, cs_ref, out_ref):
    cs = cs_ref[...]
    for b in range(BS):
        x = xt_ref[:, b * C:(b + 1) * C]
        l = lap_ref[:, b * C:(b + 1) * C]
        acc = (jnp.dot(x, cs[0:C], preferred_element_type=jnp.float32)
               + jnp.dot(l, cs[C:2 * C], preferred_element_type=jnp.float32))
        out_ref[b] = acc.T


def _combine2_body(part_ref, gv_ref, cs_ref, out_ref):
    cs = cs_ref[...]
    for b in range(BS):
        e = gv_ref[:, b * C:(b + 1) * C]
        n = gv_ref[:, D + b * C:D + (b + 1) * C]
        acc = (jnp.dot(e, cs[0:C], preferred_element_type=jnp.float32)
               + jnp.dot(n, cs[C:2 * C], preferred_element_type=jnp.float32))
        out_ref[b] = part_ref[b] + acc.T


def _tc_combine1(xtp, lap, cs01):
    return pl.pallas_call(
        _combine1_body,
        grid=(NVP // _BLK,),
        in_specs=[
            pl.BlockSpec((_BLK, D), lambda i: (i, 0)),
            pl.BlockSpec((_BLK, D), lambda i: (i, 0)),
            pl.BlockSpec((2 * C, C), lambda i: (0, 0)),
        ],
        out_specs=pl.BlockSpec((BS, C, _BLK), lambda i: (0, 0, i)),
        out_shape=jax.ShapeDtypeStruct((BS, C, NVP), jnp.float32),
    )(xtp, lap, cs01)


def _tc_combine2(part, gv, cs23):
    return pl.pallas_call(
        _combine2_body,
        grid=(NVP // _BLK,),
        in_specs=[
            pl.BlockSpec((BS, C, _BLK), lambda i: (0, 0, i)),
            pl.BlockSpec((_BLK, 2 * D), lambda i: (i, 0)),
            pl.BlockSpec((2 * C, C), lambda i: (0, 0)),
        ],
        out_specs=pl.BlockSpec((BS, C, _BLK), lambda i: (0, 0, i)),
        out_shape=jax.ShapeDtypeStruct((BS, C, NV), jnp.float32),
    )(part, gv, cs23)


def kernel(input, coeffs, G_rows, G_cols, G_vals, L_rows, L_cols, L_vals,
           F_rows, F_cols, F_vals, NS, EW):
    f32 = jnp.float32
    xpad = jnp.concatenate(
        [input, jnp.zeros((BS, C, 510), dtype=input.dtype)], axis=-1)
    xtp = _tc_xtp(xpad)

    gcols9 = G_cols.reshape(3, NF, 3).transpose(1, 0, 2).reshape(NF, 9)
    gvals9 = G_vals.reshape(3, NF, 3).transpose(1, 0, 2).reshape(NF, 9)
    gidx = gcols9.reshape(NF // FCH, FCH * 9)
    gw = jnp.concatenate(
        [gvals9, EW, NS, jnp.zeros((NF, 1), f32)],
        axis=1).reshape(NF // FCH, FCH * 16)

    zpad = NVP - NV
    lidx = jnp.concatenate(
        [L_cols.reshape(NV, 7), jnp.zeros((zpad, 7), jnp.int32)], axis=0
    ).reshape(NVP // LCH, LCH * 7)
    lw = jnp.concatenate(
        [L_vals.reshape(NV, 7), jnp.zeros((NV, 9), f32)], axis=1)
    lw = jnp.concatenate([lw, jnp.zeros((zpad, 16), f32)], axis=0
                         ).reshape(NVP // LCH, LCH * 16)

    fidx = jnp.concatenate(
        [F_cols.reshape(NV, 6), jnp.zeros((zpad, 6), jnp.int32)], axis=0
    ).reshape(NVP // VCH, VCH * 6)
    fw = jnp.concatenate(
        [F_vals.reshape(NV, 6), jnp.zeros((NV, 10), f32)], axis=1)
    fw = jnp.concatenate([fw, jnp.zeros((zpad, 16), f32)], axis=0
                         ).reshape(NVP // VCH, VCH * 16)

    cstack = coeffs.reshape(C, 4, C).transpose(1, 0, 2).reshape(4 * C, C)
    cs01 = cstack[0:2 * C]
    cs23 = cstack[2 * C:4 * C]

    faces2 = _sc_face(xtp, gidx, gw)
    lap = _sc_lap(xtp, lidx, lw)
    part = _tc_combine1(xtp, lap, cs01)
    gv = _sc_f2v(faces2, fidx, fw)
    return _tc_combine2(part, gv, cs23)


# speedup vs baseline: 1.9078x; 1.3369x over previous
import functools

import jax
import jax.numpy as jnp
from jax import lax
from jax.experimental import pallas as pl
from jax.experimental.pallas import tpu as pltpu
from jax.experimental.pallas import tpu_sc as plsc

NV = 40962
NV_PREV = 10242
NF = 81920
BS = 2
C = 128
D = BS * C

_NC, _NSC = 2, 16
NW = _NC * _NSC
NVP = 41472
VPW = NVP // NW

FCH = 8
FNCH = (NF // NW) // FCH
LCH = 8
LNCH = VPW // LCH
VCH = 8
VNCH = VPW // VCH

_MESH = dict(core_axis_name="c", subcore_axis_name="s",
             num_cores=_NC, num_subcores=_NSC)


def _wid():
    return lax.axis_index("s") * _NC + lax.axis_index("c")


def _emit_pass(idx_src, w_src, table, out_hbm, idxb, wbs, gb, ob, sems,
               rch, nch, compute_rows):
    smi = sems[0:2]
    smw = sems[2:4]
    smg = sems[4:6]
    smo = sems[6:8]
    c0 = _wid() * nch

    def issue_idx(c, s, sem_slot):
        for src, dst in zip(idx_src(c), idxb[s]):
            pltpu.async_copy(src, dst, smi[sem_slot])

    def wait_idx(c, s, sem_slot):
        for src, dst in zip(idx_src(c), idxb[s]):
            pltpu.make_async_copy(src, dst, smi[sem_slot]).wait()

    def issue_w(c, s):
        for src, dst in zip(w_src(c), wbs[s]):
            pltpu.async_copy(src, dst, smw[s])

    def wait_w(c, s):
        for src, dst in zip(w_src(c), wbs[s]):
            pltpu.make_async_copy(src, dst, smw[s]).wait()

    def issue_gather(s):
        for idxv, gdst in zip(idxb[s], gb[s]):
            pltpu.async_copy(table.at[idxv], gdst, smg[s])

    def wait_gather(s):
        for idxv, gdst in zip(idxb[s], gb[s]):
            pltpu.make_async_copy(table.at[idxv], gdst, smg[s]).wait()

    issue_idx(c0, 0, 0)
    wait_idx(c0, 0, 0)
    issue_idx(c0 + 1, 1, 1)
    issue_w(c0, 0)
    issue_gather(0)

    def pair(p, carry):
        for b in range(2):
            ci = 2 * p + b
            c = c0 + ci
            s, s1 = b, 1 - b
            wait_gather(s)

            @pl.when(ci + 2 < nch)
            def _():
                issue_idx(c + 2, s, s)

            @pl.when(ci + 1 < nch)
            def _():
                wait_idx(c + 1, s1, s1)
                issue_w(c + 1, s1)
                issue_gather(s1)

            wait_w(c, s)

            @pl.when(ci >= 2)
            def _():
                pltpu.make_async_copy(
                    ob[s], out_hbm.at[pl.ds(c * rch, rch)], smo[s]).wait()

            compute_rows(gb[s], wbs[s], ob[s])
            pltpu.async_copy(ob[s], out_hbm.at[pl.ds(c * rch, rch)], smo[s])
        return carry

    lax.fori_loop(0, nch // 2, pair, 0)
    pltpu.make_async_copy(ob[0], out_hbm.at[pl.ds(c0 * rch, rch)], smo[0]).wait()
    pltpu.make_async_copy(ob[1], out_hbm.at[pl.ds(c0 * rch, rch)], smo[1]).wait()


def _face_rows(gbs, wvs, ob):
    g0b, g1b, g2b = gbs
    w0b, w1b, w2b, ewb, nsb = wvs

    @plsc.parallel_loop(0, FCH)
    def face(fb):
        g0 = fb * 3
        wv0 = w0b[pl.ds(g0, 16)]
        wv1 = w1b[pl.ds(g0, 16)]
        wv2 = w2b[pl.ds(g0, 16)]
        ev = ewb[pl.ds(g0, 16)]
        nv = nsb[pl.ds(g0, 16)]
        for t in range(D // 16):
            sl = pl.ds(t * 16, 16)
            s0 = wv0[0] * g0b[g0, sl] + wv0[1] * g0b[g0 + 1, sl] + wv0[2] * g0b[g0 + 2, sl]
            s1 = wv1[0] * g1b[g0, sl] + wv1[1] * g1b[g0 + 1, sl] + wv1[2] * g1b[g0 + 2, sl]
            s2 = wv2[0] * g2b[g0, sl] + wv2[1] * g2b[g0 + 1, sl] + wv2[2] * g2b[g0 + 2, sl]
            ob[fb, sl] = ev[0] * s0 + ev[1] * s1 + ev[2] * s2
            ob[fb, pl.ds(D + t * 16, 16)] = nv[0] * s0 + nv[1] * s1 + nv[2] * s2


def _lap_rows(gbs, wvs, ob):
    gb = gbs[0]
    wvb = wvs[0]

    @plsc.parallel_loop(0, LCH)
    def vert(vb):
        g0 = vb * 7
        wv = wvb[pl.ds(g0, 16)]
        w = [wv[j] for j in range(7)]
        for t in range(D // 16):
            sl = pl.ds(t * 16, 16)
            acc = w[0] * gb[g0 + 0, sl]
            for j in range(1, 7):
                acc = acc + w[j] * gb[g0 + j, sl]
            ob[vb, sl] = acc


def _f2v_rows(gbs, wvs, ob):
    gb = gbs[0]
    wvb = wvs[0]

    @plsc.parallel_loop(0, VCH)
    def vert(vb):
        g0 = vb * 6
        wv = wvb[pl.ds(g0, 16)]
        w = [wv[j] for j in range(6)]
        for t in range(2 * D // 16):
            sl = pl.ds(t * 16, 16)
            acc = w[0] * gb[g0 + 0, sl]
            for j in range(1, 6):
                acc = acc + w[j] * gb[g0 + j, sl]
            ob[vb, sl] = acc


def _face_body(xt, gcols, gvals, ewf, nsf, faces2, *scr):
    idxb = (scr[0:3], scr[3:6])
    wbs = (scr[6:11], scr[11:16])
    gb = (scr[16:19], scr[19:22])
    ob = scr[22:24]
    sems = scr[24:32]

    def idx_src(c):
        return [gcols.at[pl.ds(d * 3 * NF + c * 24, 24)] for d in range(3)]

    def w_src(c):
        return ([gvals.at[pl.ds(d * 3 * NF + c * 24, 40)] for d in range(3)]
                + [ewf.at[pl.ds(c * 24, 40)], nsf.at[pl.ds(c * 24, 40)]])

    _emit_pass(idx_src, w_src, xt, faces2, idxb, wbs, gb, ob, sems,
               FCH, FNCH, _face_rows)


def _sc_face(xt, gcols, gvals, ewf, nsf):
    scr = (
        [pltpu.VMEM((24,), jnp.int32)] * 6
        + [pltpu.VMEM((40,), jnp.float32)] * 10
        + [pltpu.VMEM((24, D), jnp.float32)] * 6
        + [pltpu.VMEM((FCH, 2 * D), jnp.float32)] * 2
        + [pltpu.SemaphoreType.DMA] * 8
    )
    return pl.kernel(
        _face_body,
        out_type=jax.ShapeDtypeStruct((NF, 2 * D), jnp.float32),
        mesh=plsc.VectorSubcoreMesh(**_MESH),
        scratch_types=scr,
    )(xt, gcols, gvals, ewf, nsf)


def _vert_body_maker(k, rch, nch, rows_fn, wfetch):
    def body(table, cols, vals, out, *scr):
        idxb = (scr[0:1], scr[1:2])
        wbs = (scr[2:3], scr[3:4])
        gb = (scr[4:5], scr[5:6])
        ob = scr[6:8]
        sems = scr[8:16]

        def idx_src(c):
            return [cols.at[pl.ds(c * (rch * k), rch * k)]]

        def w_src(c):
            return [vals.at[pl.ds(c * (rch * k), wfetch)]]

        _emit_pass(idx_src, w_src, table, out, idxb, wbs, gb, ob, sems,
                   rch, nch, rows_fn)

    return body


def _sc_lap(xt, lcols, lvals):
    scr = (
        [pltpu.VMEM((LCH * 7,), jnp.int32)] * 2
        + [pltpu.VMEM((72,), jnp.float32)] * 2
        + [pltpu.VMEM((LCH * 7, D), jnp.float32)] * 2
        + [pltpu.VMEM((LCH, D), jnp.float32)] * 2
        + [pltpu.SemaphoreType.DMA] * 8
    )
    return pl.kernel(
        _vert_body_maker(7, LCH, LNCH, _lap_rows, 72),
        out_type=jax.ShapeDtypeStruct((NVP, D), jnp.float32),
        mesh=plsc.VectorSubcoreMesh(**_MESH),
        scratch_types=scr,
    )(xt, lcols, lvals)


def _sc_f2v(faces2, fcols, fvals):
    scr = (
        [pltpu.VMEM((VCH * 6,), jnp.int32)] * 2
        + [pltpu.VMEM((64,), jnp.float32)] * 2
        + [pltpu.VMEM((VCH * 6, 2 * D), jnp.float32)] * 2
        + [pltpu.VMEM((VCH, 2 * D), jnp.float32)] * 2
        + [pltpu.SemaphoreType.DMA] * 8
    )
    return pl.kernel(
        _vert_body_maker(6, VCH, VNCH, _f2v_rows, 64),
        out_type=jax.ShapeDtypeStruct((NVP, 2 * D), jnp.float32),
        mesh=plsc.VectorSubcoreMesh(**_MESH),
        scratch_types=scr,
    )(faces2, fcols, fvals)


_BLK = 512


def _xtp_body(xp_ref, out_ref):
    i = pl.program_id(0)
    base = jnp.minimum(i, (NV_PREV // _BLK)) * _BLK
    valid = (i * _BLK + lax.broadcasted_iota(jnp.int32, (_BLK, C), 0)) < NV_PREV
    for b in range(BS):
        vals = xp_ref[b, :, pl.ds(base, _BLK)].T
        out_ref[:, b * C:(b + 1) * C] = jnp.where(valid, vals, 1.0)


def _tc_xtp(xpad):
    return pl.pallas_call(
        _xtp_body,
        grid=(NVP // _BLK,),
        in_specs=[pl.BlockSpec((BS, C, NV_PREV + (_BLK - NV_PREV % _BLK)),
                               lambda i: (0, 0, 0))],
        out_specs=pl.BlockSpec((_BLK, D), lambda i: (i, 0)),
        out_shape=jax.ShapeDtypeStruct((NVP, D), jnp.float32),
    )(xpad)


def _combine1_body(xt_ref, lap_ref, cs_ref, out_ref):
    cs = cs_ref[...]
    for b in range(BS):
        x = xt_ref[:, b * C:(b + 1) * C]
        l = lap_ref[:, b * C:(b + 1) * C]
        acc = (jnp.dot(x, cs[0:C], preferred_element_type=jnp.float32)
               + jnp.dot(l, cs[C:2 * C], preferred_element_type=jnp.float32))
        out_ref[b] = acc.T


def _combine2_body(part_ref, gv_ref, cs_ref, out_ref):
    cs = cs_ref[...]
    for b in range(BS):
        e = gv_ref[:, b * C:(b + 1) * C]
        n = gv_ref[:, D + b * C:D + (b + 1) * C]
        acc = (jnp.dot(e, cs[0:C], preferred_element_type=jnp.float32)
               + jnp.dot(n, cs[C:2 * C], preferred_element_type=jnp.float32))
        out_ref[b] = part_ref[b] + acc.T


def _tc_combine1(xtp, lap, cs01):
    return pl.pallas_call(
        _combine1_body,
        grid=(NVP // _BLK,),
        in_specs=[
            pl.BlockSpec((_BLK, D), lambda i: (i, 0)),
            pl.BlockSpec((_BLK, D), lambda i: (i, 0)),
            pl.BlockSpec((2 * C, C), lambda i: (0, 0)),
        ],
        out_specs=pl.BlockSpec((BS, C, _BLK), lambda i: (0, 0, i)),
        out_shape=jax.ShapeDtypeStruct((BS, C, NVP), jnp.float32),
    )(xtp, lap, cs01)


def _tc_combine2(part, gv, cs23):
    return pl.pallas_call(
        _combine2_body,
        grid=(NVP // _BLK,),
        in_specs=[
            pl.BlockSpec((BS, C, _BLK), lambda i: (0, 0, i)),
            pl.BlockSpec((_BLK, 2 * D), lambda i: (i, 0)),
            pl.BlockSpec((2 * C, C), lambda i: (0, 0)),
        ],
        out_specs=pl.BlockSpec((BS, C, _BLK), lambda i: (0, 0, i)),
        out_shape=jax.ShapeDtypeStruct((BS, C, NV), jnp.float32),
    )(part, gv, cs23)


def _pad1d(a, n, dtype):
    return jnp.concatenate([a.reshape(-1), jnp.zeros((n - a.size,), dtype)])


def kernel(input, coeffs, G_rows, G_cols, G_vals, L_rows, L_cols, L_vals,
           F_rows, F_cols, F_vals, NS, EW):
    f32 = jnp.float32
    i32 = jnp.int32
    xpad = jnp.concatenate(
        [input, jnp.zeros((BS, C, 510), dtype=input.dtype)], axis=-1)
    xtp = _tc_xtp(xpad)

    gvalsf = _pad1d(G_vals, 3 * 3 * NF + 40, f32)
    ewf = _pad1d(EW, 3 * NF + 40, f32)
    nsf = _pad1d(NS, 3 * NF + 40, f32)

    lcols = _pad1d(L_cols, NVP * 7, i32)
    lvals = _pad1d(L_vals, NVP * 7 + 72, f32)
    fcols = _pad1d(F_cols, NVP * 6, i32)
    fvals = _pad1d(F_vals, NVP * 6 + 64, f32)

    cstack = coeffs.reshape(C, 4, C).transpose(1, 0, 2).reshape(4 * C, C)
    cs01 = cstack[0:2 * C]
    cs23 = cstack[2 * C:4 * C]

    faces2 = _sc_face(xtp, G_cols, gvalsf, ewf, nsf)
    lap = _sc_lap(xtp, lcols, lvals)
    part = _tc_combine1(xtp, lap, cs01)
    gv = _sc_f2v(faces2, fcols, fvals)
    return _tc_combine2(part, gv, cs23)


# speedup vs baseline: 2.0831x; 1.0919x over previous
import functools

import jax
import jax.numpy as jnp
from jax import lax
from jax.experimental import pallas as pl
from jax.experimental.pallas import tpu as pltpu
from jax.experimental.pallas import tpu_sc as plsc

NV = 40962
NV_PREV = 10242
NF = 81920
BS = 2
C = 128
D = BS * C

_NC, _NSC = 2, 16
NW = _NC * _NSC
NVP = 41472
VPW = NVP // NW

FCH = 16
FNCH = (NF // NW) // FCH
LCH = 8
LNCH = VPW // LCH
VCH = 8
VNCH = VPW // VCH

_MESH = dict(core_axis_name="c", subcore_axis_name="s",
             num_cores=_NC, num_subcores=_NSC)


def _wid():
    return lax.axis_index("s") * _NC + lax.axis_index("c")


def _emit_pass(idx_src, w_src, table, out_hbm, idxb, wbs, gb, ob, sems,
               rch, nch, compute_rows):
    smi = sems[0:2]
    smw = sems[2:4]
    smg = sems[4:6]
    smo = sems[6:8]
    c0 = _wid() * nch

    def issue_idx(c, s, sem_slot):
        for src, dst in zip(idx_src(c), idxb[s]):
            pltpu.async_copy(src, dst, smi[sem_slot])

    def wait_idx(c, s, sem_slot):
        for src, dst in zip(idx_src(c), idxb[s]):
            pltpu.make_async_copy(src, dst, smi[sem_slot]).wait()

    def issue_w(c, s):
        for src, dst in zip(w_src(c), wbs[s]):
            pltpu.async_copy(src, dst, smw[s])

    def wait_w(c, s):
        for src, dst in zip(w_src(c), wbs[s]):
            pltpu.make_async_copy(src, dst, smw[s]).wait()

    def issue_gather(s):
        for idxv, gdst in zip(idxb[s], gb[s]):
            pltpu.async_copy(table.at[idxv], gdst, smg[s])

    def wait_gather(s):
        for idxv, gdst in zip(idxb[s], gb[s]):
            pltpu.make_async_copy(table.at[idxv], gdst, smg[s]).wait()

    issue_idx(c0, 0, 0)
    wait_idx(c0, 0, 0)
    issue_idx(c0 + 1, 1, 1)
    issue_w(c0, 0)
    issue_gather(0)

    def pair(p, carry):
        for b in range(2):
            ci = 2 * p + b
            c = c0 + ci
            s, s1 = b, 1 - b
            wait_gather(s)

            @pl.when(ci + 2 < nch)
            def _():
                issue_idx(c + 2, s, s)

            @pl.when(ci + 1 < nch)
            def _():
                wait_idx(c + 1, s1, s1)
                issue_w(c + 1, s1)
                issue_gather(s1)

            wait_w(c, s)

            @pl.when(ci >= 2)
            def _():
                pltpu.make_async_copy(
                    ob[s], out_hbm.at[pl.ds(c * rch, rch)], smo[s]).wait()

            compute_rows(gb[s], wbs[s], ob[s])
            pltpu.async_copy(ob[s], out_hbm.at[pl.ds(c * rch, rch)], smo[s])
        return carry

    lax.fori_loop(0, nch // 2, pair, 0)
    pltpu.make_async_copy(ob[0], out_hbm.at[pl.ds(c0 * rch, rch)], smo[0]).wait()
    pltpu.make_async_copy(ob[1], out_hbm.at[pl.ds(c0 * rch, rch)], smo[1]).wait()


def _face_rows(gbs, wvs, ob):
    g0b, g1b, g2b = gbs
    w0b, w1b, w2b, ewb, nsb = wvs

    @plsc.parallel_loop(0, FCH)
    def face(fb):
        g0 = fb * 3
        wv0 = w0b[pl.ds(g0, 16)]
        wv1 = w1b[pl.ds(g0, 16)]
        wv2 = w2b[pl.ds(g0, 16)]
        ev = ewb[pl.ds(g0, 16)]
        nv = nsb[pl.ds(g0, 16)]
        for t in range(D // 16):
            sl = pl.ds(t * 16, 16)
            s0 = wv0[0] * g0b[g0, sl] + wv0[1] * g0b[g0 + 1, sl] + wv0[2] * g0b[g0 + 2, sl]
            s1 = wv1[0] * g1b[g0, sl] + wv1[1] * g1b[g0 + 1, sl] + wv1[2] * g1b[g0 + 2, sl]
            s2 = wv2[0] * g2b[g0, sl] + wv2[1] * g2b[g0 + 1, sl] + wv2[2] * g2b[g0 + 2, sl]
            ob[fb, sl] = ev[0] * s0 + ev[1] * s1 + ev[2] * s2
            ob[fb, pl.ds(D + t * 16, 16)] = nv[0] * s0 + nv[1] * s1 + nv[2] * s2


def _lap_rows(gbs, wvs, ob):
    gb = gbs[0]
    wvb = wvs[0]

    @plsc.parallel_loop(0, LCH)
    def vert(vb):
        g0 = vb * 7
        wv = wvb[pl.ds(g0, 16)]
        w = [wv[j] for j in range(7)]
        for t in range(D // 16):
            sl = pl.ds(t * 16, 16)
            acc = w[0] * gb[g0 + 0, sl]
            for j in range(1, 7):
                acc = acc + w[j] * gb[g0 + j, sl]
            ob[vb, sl] = acc


def _f2v_rows(gbs, wvs, ob):
    gb = gbs[0]
    wvb = wvs[0]

    @plsc.parallel_loop(0, VCH)
    def vert(vb):
        g0 = vb * 6
        wv = wvb[pl.ds(g0, 16)]
        w = [wv[j] for j in range(6)]
        for t in range(2 * D // 16):
            sl = pl.ds(t * 16, 16)
            acc = w[0] * gb[g0 + 0, sl]
            for j in range(1, 6):
                acc = acc + w[j] * gb[g0 + j, sl]
            ob[vb, sl] = acc


def _face_body(xt, gcols, gvals, ewf, nsf, faces2, *scr):
    idxb = (scr[0:3], scr[3:6])
    wbs = (scr[6:11], scr[11:16])
    gb = (scr[16:19], scr[19:22])
    ob = scr[22:24]
    sems = scr[24:32]

    def idx_src(c):
        return [gcols.at[pl.ds(d * 3 * NF + c * 48, 48)] for d in range(3)]

    def w_src(c):
        return ([gvals.at[pl.ds(d * 3 * NF + c * 48, 64)] for d in range(3)]
                + [ewf.at[pl.ds(c * 48, 64)], nsf.at[pl.ds(c * 48, 64)]])

    _emit_pass(idx_src, w_src, xt, faces2, idxb, wbs, gb, ob, sems,
               FCH, FNCH, _face_rows)


def _sc_face(xt, gcols, gvals, ewf, nsf):
    scr = (
        [pltpu.VMEM((48,), jnp.int32)] * 6
        + [pltpu.VMEM((64,), jnp.float32)] * 10
        + [pltpu.VMEM((48, D), jnp.float32)] * 6
        + [pltpu.VMEM((FCH, 2 * D), jnp.float32)] * 2
        + [pltpu.SemaphoreType.DMA] * 8
    )
    return pl.kernel(
        _face_body,
        out_type=jax.ShapeDtypeStruct((NF, 2 * D), jnp.float32),
        mesh=plsc.VectorSubcoreMesh(**_MESH),
        scratch_types=scr,
    )(xt, gcols, gvals, ewf, nsf)


def _vert_body_maker(k, rch, nch, rows_fn, wfetch):
    def body(table, cols, vals, out, *scr):
        idxb = (scr[0:1], scr[1:2])
        wbs = (scr[2:3], scr[3:4])
        gb = (scr[4:5], scr[5:6])
        ob = scr[6:8]
        sems = scr[8:16]

        def idx_src(c):
            return [cols.at[pl.ds(c * (rch * k), rch * k)]]

        def w_src(c):
            return [vals.at[pl.ds(c * (rch * k), wfetch)]]

        _emit_pass(idx_src, w_src, table, out, idxb, wbs, gb, ob, sems,
                   rch, nch, rows_fn)

    return body


def _sc_lap(xt, lcols, lvals):
    scr = (
        [pltpu.VMEM((LCH * 7,), jnp.int32)] * 2
        + [pltpu.VMEM((72,), jnp.float32)] * 2
        + [pltpu.VMEM((LCH * 7, D), jnp.float32)] * 2
        + [pltpu.VMEM((LCH, D), jnp.float32)] * 2
        + [pltpu.SemaphoreType.DMA] * 8
    )
    return pl.kernel(
        _vert_body_maker(7, LCH, LNCH, _lap_rows, 72),
        out_type=jax.ShapeDtypeStruct((NVP, D), jnp.float32),
        mesh=plsc.VectorSubcoreMesh(**_MESH),
        scratch_types=scr,
    )(xt, lcols, lvals)


def _sc_f2v(faces2, fcols, fvals):
    scr = (
        [pltpu.VMEM((VCH * 6,), jnp.int32)] * 2
        + [pltpu.VMEM((64,), jnp.float32)] * 2
        + [pltpu.VMEM((VCH * 6, 2 * D), jnp.float32)] * 2
        + [pltpu.VMEM((VCH, 2 * D), jnp.float32)] * 2
        + [pltpu.SemaphoreType.DMA] * 8
    )
    return pl.kernel(
        _vert_body_maker(6, VCH, VNCH, _f2v_rows, 64),
        out_type=jax.ShapeDtypeStruct((NVP, 2 * D), jnp.float32),
        mesh=plsc.VectorSubcoreMesh(**_MESH),
        scratch_types=scr,
    )(faces2, fcols, fvals)


_BLK = 512


def _xtp_body(xp_ref, out_ref):
    i = pl.program_id(0)
    base = jnp.minimum(i, (NV_PREV // _BLK)) * _BLK
    valid = (i * _BLK + lax.broadcasted_iota(jnp.int32, (_BLK, C), 0)) < NV_PREV
    for b in range(BS):
        vals = xp_ref[b, :, pl.ds(base, _BLK)].T
        out_ref[:, b * C:(b + 1) * C] = jnp.where(valid, vals, 1.0)


def _tc_xtp(xpad):
    return pl.pallas_call(
        _xtp_body,
        grid=(NVP // _BLK,),
        in_specs=[pl.BlockSpec((BS, C, NV_PREV + (_BLK - NV_PREV % _BLK)),
                               lambda i: (0, 0, 0))],
        out_specs=pl.BlockSpec((_BLK, D), lambda i: (i, 0)),
        out_shape=jax.ShapeDtypeStruct((NVP, D), jnp.float32),
    )(xpad)


def _combine1_body(xt_ref, lap_ref, cs_ref, out_ref):
    cs = cs_ref[...]
    for b in range(BS):
        x = xt_ref[:, b * C:(b + 1) * C]
        l = lap_ref[:, b * C:(b + 1) * C]
        acc = (jnp.dot(x, cs[0:C], preferred_element_type=jnp.float32)
               + jnp.dot(l, cs[C:2 * C], preferred_element_type=jnp.float32))
        out_ref[b] = acc.T


def _combine2_body(part_ref, gv_ref, cs_ref, out_ref):
    cs = cs_ref[...]
    for b in range(BS):
        e = gv_ref[:, b * C:(b + 1) * C]
        n = gv_ref[:, D + b * C:D + (b + 1) * C]
        acc = (jnp.dot(e, cs[0:C], preferred_element_type=jnp.float32)
               + jnp.dot(n, cs[C:2 * C], preferred_element_type=jnp.float32))
        out_ref[b] = part_ref[b] + acc.T


def _tc_combine1(xtp, lap, cs01):
    return pl.pallas_call(
        _combine1_body,
        grid=(NVP // _BLK,),
        in_specs=[
            pl.BlockSpec((_BLK, D), lambda i: (i, 0)),
            pl.BlockSpec((_BLK, D), lambda i: (i, 0)),
            pl.BlockSpec((2 * C, C), lambda i: (0, 0)),
        ],
        out_specs=pl.BlockSpec((BS, C, _BLK), lambda i: (0, 0, i)),
        out_shape=jax.ShapeDtypeStruct((BS, C, NVP), jnp.float32),
    )(xtp, lap, cs01)


def _tc_combine2(part, gv, cs23):
    return pl.pallas_call(
        _combine2_body,
        grid=(NVP // _BLK,),
        in_specs=[
            pl.BlockSpec((BS, C, _BLK), lambda i: (0, 0, i)),
            pl.BlockSpec((_BLK, 2 * D), lambda i: (i, 0)),
            pl.BlockSpec((2 * C, C), lambda i: (0, 0)),
        ],
        out_specs=pl.BlockSpec((BS, C, _BLK), lambda i: (0, 0, i)),
        out_shape=jax.ShapeDtypeStruct((BS, C, NV), jnp.float32),
    )(part, gv, cs23)


def _pad1d(a, n, dtype):
    return jnp.concatenate([a.reshape(-1), jnp.zeros((n - a.size,), dtype)])


def kernel(input, coeffs, G_rows, G_cols, G_vals, L_rows, L_cols, L_vals,
           F_rows, F_cols, F_vals, NS, EW):
    f32 = jnp.float32
    i32 = jnp.int32
    xpad = jnp.concatenate(
        [input, jnp.zeros((BS, C, 510), dtype=input.dtype)], axis=-1)
    xtp = _tc_xtp(xpad)

    gvalsf = _pad1d(G_vals, 3 * 3 * NF + 64, f32)
    ewf = _pad1d(EW, 3 * NF + 64, f32)
    nsf = _pad1d(NS, 3 * NF + 64, f32)

    lcols = _pad1d(L_cols, NVP * 7, i32)
    lvals = _pad1d(L_vals, NVP * 7 + 72, f32)
    fcols = _pad1d(F_cols, NVP * 6, i32)
    fvals = _pad1d(F_vals, NVP * 6 + 64, f32)

    cstack = coeffs.reshape(C, 4, C).transpose(1, 0, 2).reshape(4 * C, C)
    cs01 = cstack[0:2 * C]
    cs23 = cstack[2 * C:4 * C]

    faces2 = _sc_face(xtp, G_cols, gvalsf, ewf, nsf)
    lap = _sc_lap(xtp, lcols, lvals)
    part = _tc_combine1(xtp, lap, cs01)
    gv = _sc_f2v(faces2, fcols, fvals)
    return _tc_combine2(part, gv, cs23)
